# baseline (speedup 1.0000x reference)
import jax
import jax.numpy as jnp
from jax.experimental import pallas as pl

_A = 20
_ATD = 6
_DIM = 512
_IN = 1024
_B, _H, _W = 8, 37, 37
_PW = _W + 2
_NP = 1536
_XL = _NP + 2 * _PW + 2


def _body(x_ref, w1_ref, b1_ref, w2_ref, b2_ref, o_ref):
    x = x_ref[0]
    acc = jnp.zeros((_DIM, _NP), jnp.float32)
    for t in range(9):
        off = (t // 3) * _PW + (t % 3)
        acc = acc + jnp.dot(
            w1_ref[t], x[:, off:off + _NP],
            preferred_element_type=jnp.float32)
    acc = acc + b1_ref[...]
    y = jnp.clip(acc, 0.0, 6.0).astype(jnp.bfloat16)
    z = jax.lax.dot_general(
        y, w2_ref[...], (((0,), (0,)), ((), ())),
        preferred_element_type=jnp.float32)
    o_ref[0] = z + b2_ref[...]


def kernel(fmap, W1, b1, W2, b2):
    xp = jnp.pad(fmap, ((0, 0), (0, 0), (1, 1), (1, 1)))
    xf = xp.reshape(_B, _IN, (_H + 2) * _PW)
    xf = jnp.pad(xf, ((0, 0), (0, 0), (0, _XL - (_H + 2) * _PW)))
    xf = xf.astype(jnp.bfloat16)

    w1 = jnp.transpose(W1, (2, 3, 0, 1)).reshape(9, _DIM, _IN)
    w1 = w1.astype(jnp.bfloat16)
    w2 = W2.reshape(_A * _ATD, _DIM).T.astype(jnp.bfloat16)
    b1c = b1.reshape(_DIM, 1)
    b2c = b2.reshape(1, _A * _ATD)

    out = pl.pallas_call(
        _body,
        grid=(_B,),
        in_specs=[
            pl.BlockSpec((1, _IN, _XL), lambda b: (b, 0, 0)),
            pl.BlockSpec((9, _DIM, _IN), lambda b: (0, 0, 0)),
            pl.BlockSpec((_DIM, 1), lambda b: (0, 0)),
            pl.BlockSpec((_DIM, _A * _ATD), lambda b: (0, 0)),
            pl.BlockSpec((1, _A * _ATD), lambda b: (0, 0)),
        ],
        out_specs=pl.BlockSpec((1, _NP, _A * _ATD), lambda b: (b, 0, 0)),
        out_shape=jax.ShapeDtypeStruct((_B, _NP, _A * _ATD), jnp.float32),
    )(xf, w1, b1c, w2, b2c)

    out = out[:, :_H * _PW, :].reshape(_B, _H, _PW, _A * _ATD)
    out = out[:, :, :_W, :].reshape(_B, _H, _W, _A, _ATD)
    return out
